# baseline (speedup 1.0000x reference)
import functools

import jax
import jax.numpy as jnp
from jax import lax
from jax.experimental import pallas as pl
from jax.experimental.pallas import tpu as pltpu
from jax.experimental.pallas import tpu_sc as plsc

_NBARS = 100
_STRIDE = 101
_NW = 32
_TPW = 1024
_CHUNK = 256
_L = 16
_LN2 = 0.6931471805599453

_B = 4
_T = 8192


def _ln(x):
    bits = lax.bitcast_convert_type(x, jnp.int32)
    e = ((bits >> 23) & 255) - 127
    m = lax.bitcast_convert_type((bits & 0x007FFFFF) | 0x3F800000, jnp.float32)
    big = m > 1.4142135
    m = jnp.where(big, m * 0.5, m)
    ef = (e + jnp.where(big, 1, 0)).astype(jnp.float32)
    t = (m - 1.0) / (m + 1.0)
    t2 = t * t
    p = 1.0 + t2 * (0.3333333333 + t2 * (0.2 + t2 * (0.1428571429 + t2 * 0.1111111111)))
    return ef * _LN2 + (2.0 * t) * p


def _bc(v):
    return jnp.full((_L,), v, jnp.int32)


def _sc_body(a_hbm, y_hbm, borders_hbm, out_hbm,
             buf0, buf1, y_v, out_v, borders_v, sem0, sem1):
    wid = lax.axis_index("s") * 2 + lax.axis_index("c")
    tok0 = wid * _TPW
    bidx = wid // (_T // _TPW)
    t0b = (wid % (_T // _TPW)) * _TPW
    bufs = (buf0, buf1)
    sems = (sem0, sem1)
    cps = [pltpu.async_copy(
        a_hbm.at[pl.ds(tok0, _CHUNK), :], buf0, sem0)]
    pltpu.sync_copy(borders_hbm, borders_v)
    pltpu.sync_copy(y_hbm.at[bidx, pl.ds(t0b, _TPW)], y_v)
    lanes = lax.iota(jnp.int32, _L)
    nchunks = _TPW // _CHUNK

    for cc in range(nchunks):
        buf = bufs[cc % 2]
        if cc + 1 < nchunks:
            cps.append(pltpu.async_copy(
                a_hbm.at[pl.ds(tok0 + (cc + 1) * _CHUNK, _CHUNK), :],
                bufs[(cc + 1) % 2], sems[(cc + 1) % 2]))
        cps[cc].wait()

        def group(g, carry):
            lrow = g * _L + lanes

            def p1(i, accs):
                c0 = i * 10
                xs = [plsc.load_gather(buf, [lrow, _bc(c0 + j)]) for j in range(10)]
                a = list(accs)
                for j in range(10):
                    a[j % 4] = jnp.maximum(a[j % 4], xs[j])
                return tuple(a)

            neg = jnp.full((_L,), -3.0e38, jnp.float32)
            m4 = lax.fori_loop(0, 10, p1, (neg, neg, neg, neg))
            m = jnp.maximum(jnp.maximum(m4[0], m4[1]), jnp.maximum(m4[2], m4[3]))

            def p2(i, accs):
                c0 = i * 10
                xs = [plsc.load_gather(buf, [lrow, _bc(c0 + j)]) for j in range(10)]
                a = list(accs)
                for j in range(10):
                    a[j % 4] = a[j % 4] + jnp.exp(xs[j] - m)
                return tuple(a)

            z = jnp.zeros((_L,), jnp.float32)
            s4 = lax.fori_loop(0, 10, p2, (z, z, z, z))
            s = (s4[0] + s4[1]) + (s4[2] + s4[3])
            lse = m + _ln(s)

            gtok = cc * _CHUNK + g * _L + lanes
            yv = plsc.load_gather(y_v, [gtok])
            idx = jnp.clip((yv * float(_NBARS)).astype(jnp.int32), 0, _NBARS - 1)
            for _ in range(2):
                blo = plsc.load_gather(borders_v, [idx])
                bhi = plsc.load_gather(borders_v, [idx + 1])
                idx = idx - jnp.where(yv <= blo, 1, 0) + jnp.where(yv > bhi, 1, 0)
                idx = jnp.clip(idx, 0, _NBARS - 1)
            blo = plsc.load_gather(borders_v, [idx])
            bhi = plsc.load_gather(borders_v, [idx + 1])
            gl = plsc.load_gather(buf, [lrow, idx])
            nll = lse - gl + _ln(bhi - blo)
            plsc.store_scatter(out_v, [gtok], nll)
            return carry

        lax.fori_loop(0, _CHUNK // _L, group, 0)

    pltpu.sync_copy(out_v, out_hbm.at[bidx, pl.ds(t0b, _TPW)])


@functools.partial(
    pl.kernel,
    mesh=plsc.VectorSubcoreMesh(core_axis_name="c", subcore_axis_name="s"),
    compiler_params=pltpu.CompilerParams(
        needs_layout_passes=False, use_tc_tiling_on_sc=False),
    out_type=jax.ShapeDtypeStruct((_B, _T), jnp.float32),
    scratch_types=[
        pltpu.VMEM((_CHUNK, _NBARS), jnp.float32),
        pltpu.VMEM((_CHUNK, _NBARS), jnp.float32),
        pltpu.VMEM((_TPW,), jnp.float32),
        pltpu.VMEM((_TPW,), jnp.float32),
        pltpu.VMEM((_NBARS + 1,), jnp.float32),
        pltpu.SemaphoreType.DMA,
        pltpu.SemaphoreType.DMA,
    ],
)
def _sc_nll(a_hbm, y_hbm, borders_hbm, out_hbm,
            buf0, buf1, y_v, out_v, borders_v, sem0, sem1):
    _sc_body(a_hbm, y_hbm, borders_hbm, out_hbm,
             buf0, buf1, y_v, out_v, borders_v, sem0, sem1)


def kernel(logits, y, borders):
    b, t, nb = logits.shape
    n = b * t
    return _sc_nll(logits.reshape(n, nb), y, borders)


# speedup vs baseline: 1.1778x; 1.1778x over previous
import functools

import jax
import jax.numpy as jnp
from jax import lax
from jax.experimental import pallas as pl
from jax.experimental.pallas import tpu as pltpu
from jax.experimental.pallas import tpu_sc as plsc

_NBARS = 100
_NW = 32
_TPW = 1024
_CHUNK = 256
_L = 16
_LN2 = 0.6931471805599453

_B = 4
_T = 8192


def _ln(x):
    bits = lax.bitcast_convert_type(x, jnp.int32)
    e = ((bits >> 23) & 255) - 127
    m = lax.bitcast_convert_type((bits & 0x007FFFFF) | 0x3F800000, jnp.float32)
    big = m > 1.4142135
    m = jnp.where(big, m * 0.5, m)
    ef = (e + jnp.where(big, 1, 0)).astype(jnp.float32)
    t = (m - 1.0) / (m + 1.0)
    t2 = t * t
    p = 1.0 + t2 * (0.3333333333 + t2 * (0.2 + t2 * (0.1428571429 + t2 * 0.1111111111)))
    return ef * _LN2 + (2.0 * t) * p


def _sc_body(a_hbm, y_hbm, borders_hbm, out_hbm,
             buf0, buf1, y_v, out_v, borders_v, sem0, sem1):
    wid = lax.axis_index("s") * 2 + lax.axis_index("c")
    tok0 = wid * _TPW
    bidx = wid // (_T // _TPW)
    t0b = (wid % (_T // _TPW)) * _TPW
    bufs = (buf0, buf1)
    sems = (sem0, sem1)
    cps = [pltpu.async_copy(
        a_hbm.at[pl.ds(tok0 * _NBARS, _CHUNK * _NBARS)], buf0, sem0)]
    pltpu.sync_copy(borders_hbm, borders_v)
    pltpu.sync_copy(y_hbm.at[bidx, pl.ds(t0b, _TPW)], y_v)
    lanes = lax.iota(jnp.int32, _L)
    nchunks = _TPW // _CHUNK

    for cc in range(nchunks):
        buf = bufs[cc % 2]
        if cc + 1 < nchunks:
            cps.append(pltpu.async_copy(
                a_hbm.at[pl.ds((tok0 + (cc + 1) * _CHUNK) * _NBARS,
                               _CHUNK * _NBARS)],
                bufs[(cc + 1) % 2], sems[(cc + 1) % 2]))
        cps[cc].wait()

        def group(g, carry):
            lrow = g * _L + lanes
            abase = lrow * _NBARS
            addr0 = abase + lanes

            def p1a(i, c):
                addr, a0, a1, a2, a3 = c
                a = [a0, a1, a2, a3]
                for j in range(10):
                    x = plsc.load_gather(buf, [addr])
                    a[j % 4] = jnp.maximum(a[j % 4], x)
                    addr = addr + 1
                return (addr, *a)

            def p1b(i, c):
                addr, col, a0, a1, a2, a3 = c
                a = [a0, a1, a2, a3]
                for j in range(10):
                    x = plsc.load_gather(buf, [addr])
                    a[j % 4] = jnp.maximum(a[j % 4], x)
                    col = col + 1
                    wrap = col >= _NBARS
                    col = jnp.where(wrap, col - _NBARS, col)
                    addr = jnp.where(wrap, addr + 1 - _NBARS, addr + 1)
                return (addr, col, *a)

            neg = jnp.full((_L,), -3.0e38, jnp.float32)
            c1 = lax.fori_loop(0, 8, p1a, (addr0, neg, neg, neg, neg))
            col80 = lanes + 80
            c1b = lax.fori_loop(0, 2, p1b, (c1[0], col80, *c1[1:]))
            m4 = c1b[2:]
            m = jnp.maximum(jnp.maximum(m4[0], m4[1]), jnp.maximum(m4[2], m4[3]))

            def p2a(i, c):
                addr, a0, a1, a2, a3 = c
                a = [a0, a1, a2, a3]
                for j in range(10):
                    x = plsc.load_gather(buf, [addr])
                    a[j % 4] = a[j % 4] + jnp.exp(x - m)
                    addr = addr + 1
                return (addr, *a)

            def p2b(i, c):
                addr, col, a0, a1, a2, a3 = c
                a = [a0, a1, a2, a3]
                for j in range(10):
                    x = plsc.load_gather(buf, [addr])
                    a[j % 4] = a[j % 4] + jnp.exp(x - m)
                    col = col + 1
                    wrap = col >= _NBARS
                    col = jnp.where(wrap, col - _NBARS, col)
                    addr = jnp.where(wrap, addr + 1 - _NBARS, addr + 1)
                return (addr, col, *a)

            z = jnp.zeros((_L,), jnp.float32)
            c2 = lax.fori_loop(0, 8, p2a, (addr0, z, z, z, z))
            c2b = lax.fori_loop(0, 2, p2b, (c2[0], col80, *c2[1:]))
            s4 = c2b[2:]
            s = (s4[0] + s4[1]) + (s4[2] + s4[3])
            lse = m + _ln(s)

            gtok = cc * _CHUNK + g * _L + lanes
            yv = plsc.load_gather(y_v, [gtok])
            idx = jnp.clip((yv * float(_NBARS)).astype(jnp.int32), 0, _NBARS - 1)
            for _ in range(2):
                blo = plsc.load_gather(borders_v, [idx])
                bhi = plsc.load_gather(borders_v, [idx + 1])
                idx = idx - jnp.where(yv <= blo, 1, 0) + jnp.where(yv > bhi, 1, 0)
                idx = jnp.clip(idx, 0, _NBARS - 1)
            blo = plsc.load_gather(borders_v, [idx])
            bhi = plsc.load_gather(borders_v, [idx + 1])
            gl = plsc.load_gather(buf, [abase + idx])
            nll = lse - gl + _ln(bhi - blo)
            plsc.store_scatter(out_v, [gtok], nll)
            return carry

        lax.fori_loop(0, _CHUNK // _L, group, 0)

    pltpu.sync_copy(out_v, out_hbm.at[bidx, pl.ds(t0b, _TPW)])


@functools.partial(
    pl.kernel,
    mesh=plsc.VectorSubcoreMesh(core_axis_name="c", subcore_axis_name="s"),
    compiler_params=pltpu.CompilerParams(
        needs_layout_passes=False, use_tc_tiling_on_sc=False),
    out_type=jax.ShapeDtypeStruct((_B, _T), jnp.float32),
    scratch_types=[
        pltpu.VMEM((_CHUNK * _NBARS,), jnp.float32),
        pltpu.VMEM((_CHUNK * _NBARS,), jnp.float32),
        pltpu.VMEM((_TPW,), jnp.float32),
        pltpu.VMEM((_TPW,), jnp.float32),
        pltpu.VMEM((_NBARS + 1,), jnp.float32),
        pltpu.SemaphoreType.DMA,
        pltpu.SemaphoreType.DMA,
    ],
)
def _sc_nll(a_hbm, y_hbm, borders_hbm, out_hbm,
            buf0, buf1, y_v, out_v, borders_v, sem0, sem1):
    _sc_body(a_hbm, y_hbm, borders_hbm, out_hbm,
             buf0, buf1, y_v, out_v, borders_v, sem0, sem1)


def kernel(logits, y, borders):
    b, t, nb = logits.shape
    n = b * t
    return _sc_nll(logits.reshape(n * nb), y, borders)


# speedup vs baseline: 1.4111x; 1.1981x over previous
import functools

import jax
import jax.numpy as jnp
from jax import lax
from jax.experimental import pallas as pl
from jax.experimental.pallas import tpu as pltpu
from jax.experimental.pallas import tpu_sc as plsc

_NBARS = 100
_ROWS = 101
_NW = 32
_TPW = 1024
_CHUNK = 256
_L = 16
_LN2 = 0.6931471805599453

_B = 4
_T = 8192


_RB = 2048


def _tc_body(x_ref, a_ref):
    x = x_ref[...]
    m = jnp.max(x, axis=1, keepdims=True)
    a_ref[0:_NBARS, :] = x.T
    a_ref[_NBARS:_ROWS, :] = m.T


def _tc_prep(logits2):
    n = logits2.shape[0]
    return pl.pallas_call(
        _tc_body,
        grid=(n // _RB,),
        in_specs=[pl.BlockSpec((_RB, _NBARS), lambda i: (i, 0))],
        out_specs=pl.BlockSpec((_ROWS, _RB), lambda i: (0, i)),
        out_shape=jax.ShapeDtypeStruct((_ROWS, n), jnp.float32),
    )(logits2)


def _ln(x):
    bits = lax.bitcast_convert_type(x, jnp.int32)
    e = ((bits >> 23) & 255) - 127
    m = lax.bitcast_convert_type((bits & 0x007FFFFF) | 0x3F800000, jnp.float32)
    big = m > 1.4142135
    m = jnp.where(big, m * 0.5, m)
    ef = (e + jnp.where(big, 1, 0)).astype(jnp.float32)
    t = (m - 1.0) / (m + 1.0)
    t2 = t * t
    p = 1.0 + t2 * (0.3333333333 + t2 * (0.2 + t2 * (0.1428571429 + t2 * 0.1111111111)))
    return ef * _LN2 + (2.0 * t) * p


def _sc_body(a_hbm, y_hbm, borders_hbm, out_hbm,
             buf0, buf1, y_v, out_v, borders_v, sem0, sem1):
    wid = lax.axis_index("s") * 2 + lax.axis_index("c")
    tok0 = wid * _TPW
    bidx = wid // (_T // _TPW)
    t0b = (wid % (_T // _TPW)) * _TPW
    bufs = (buf0, buf1)
    sems = (sem0, sem1)
    cps = [pltpu.async_copy(a_hbm.at[:, pl.ds(tok0, _CHUNK)], buf0, sem0)]
    pltpu.sync_copy(borders_hbm, borders_v)
    pltpu.sync_copy(y_hbm.at[bidx, pl.ds(t0b, _TPW)], y_v)
    lanes = lax.iota(jnp.int32, _L)
    nchunks = _TPW // _CHUNK

    for cc in range(nchunks):
        buf = bufs[cc % 2]
        if cc + 1 < nchunks:
            cps.append(pltpu.async_copy(
                a_hbm.at[:, pl.ds(tok0 + (cc + 1) * _CHUNK, _CHUNK)],
                bufs[(cc + 1) % 2], sems[(cc + 1) % 2]))
        cps[cc].wait()

        def group(g, carry):
            t0 = g * _L
            m = buf[_NBARS, pl.ds(t0, _L)]
            z = jnp.zeros((_L,), jnp.float32)

            @plsc.parallel_loop(0, 10, unroll=2, carry=(z, z, z, z))
            def s4(ci, accs):
                c0 = ci * 10
                xs = [buf[c0 + j, pl.ds(t0, _L)] for j in range(10)]
                a = list(accs)
                for j in range(10):
                    a[j % 4] = a[j % 4] + jnp.exp(xs[j] - m)
                return tuple(a)

            s = (s4[0] + s4[1]) + (s4[2] + s4[3])
            lse = m + _ln(s)

            gtok = cc * _CHUNK + t0 + lanes
            yv = plsc.load_gather(y_v, [gtok])
            idx = jnp.clip((yv * float(_NBARS)).astype(jnp.int32), 0, _NBARS - 1)
            for _ in range(2):
                blo = plsc.load_gather(borders_v, [idx])
                bhi = plsc.load_gather(borders_v, [idx + 1])
                idx = idx - jnp.where(yv <= blo, 1, 0) + jnp.where(yv > bhi, 1, 0)
                idx = jnp.clip(idx, 0, _NBARS - 1)
            blo = plsc.load_gather(borders_v, [idx])
            bhi = plsc.load_gather(borders_v, [idx + 1])
            gl = plsc.load_gather(buf, [idx, t0 + lanes])
            nll = lse - gl + _ln(bhi - blo)
            plsc.store_scatter(out_v, [gtok], nll)
            return carry

        lax.fori_loop(0, _CHUNK // _L, group, 0)

    pltpu.sync_copy(out_v, out_hbm.at[bidx, pl.ds(t0b, _TPW)])


@functools.partial(
    pl.kernel,
    mesh=plsc.VectorSubcoreMesh(core_axis_name="c", subcore_axis_name="s"),
    compiler_params=pltpu.CompilerParams(needs_layout_passes=False),
    out_type=jax.ShapeDtypeStruct((_B, _T), jnp.float32),
    scratch_types=[
        pltpu.VMEM((_ROWS, _CHUNK), jnp.float32),
        pltpu.VMEM((_ROWS, _CHUNK), jnp.float32),
        pltpu.VMEM((_TPW,), jnp.float32),
        pltpu.VMEM((_TPW,), jnp.float32),
        pltpu.VMEM((_NBARS + 1,), jnp.float32),
        pltpu.SemaphoreType.DMA,
        pltpu.SemaphoreType.DMA,
    ],
)
def _sc_nll(a_hbm, y_hbm, borders_hbm, out_hbm,
            buf0, buf1, y_v, out_v, borders_v, sem0, sem1):
    _sc_body(a_hbm, y_hbm, borders_hbm, out_hbm,
             buf0, buf1, y_v, out_v, borders_v, sem0, sem1)


def kernel(logits, y, borders):
    b, t, nb = logits.shape
    n = b * t
    a = _tc_prep(logits.reshape(n, nb))
    return _sc_nll(a, y, borders)
